# baseline (speedup 1.0000x reference)
import functools

import jax
import jax.numpy as jnp
from jax import lax
from jax.experimental import pallas as pl
from jax.experimental.pallas import tpu as pltpu
from jax.experimental.pallas import tpu_sc as plsc

TOP_K = 8
ROUTED_SCALING_FACTOR = 2.5
E = 64
T_TOTAL = 32768
LANES = 16
CHUNK = 512


def _tc_stage(x_ref, w_ref, b_ref, s_ref, biased_ref):
    logits = jax.lax.dot_general(
        w_ref[...], x_ref[...], (((1,), (1,)), ((), ())),
        preferred_element_type=jnp.float32,
    )
    s = jax.nn.sigmoid(logits)
    s_ref[...] = s
    biased_ref[...] = s + b_ref[...]


@functools.partial(jax.jit, static_argnames=("block_t",))
def _tc_gate(x, gate_weight, bias2d, block_t=4096):
    t, d = x.shape
    e = gate_weight.shape[0]
    return pl.pallas_call(
        _tc_stage,
        grid=(t // block_t,),
        in_specs=[
            pl.BlockSpec((block_t, d), lambda i: (i, 0)),
            pl.BlockSpec((e, d), lambda i: (0, 0)),
            pl.BlockSpec((e, 1), lambda i: (0, 0)),
        ],
        out_specs=[
            pl.BlockSpec((e, block_t), lambda i: (0, i)),
            pl.BlockSpec((e, block_t), lambda i: (0, i)),
        ],
        out_shape=[
            jax.ShapeDtypeStruct((e, t), jnp.float32),
            jax.ShapeDtypeStruct((e, t), jnp.float32),
        ],
    )(x, gate_weight, bias2d)


def _make_sc_topk():
    info = plsc.get_sparse_core_info()
    nc, ns = info.num_cores, info.num_subcores
    nw = nc * ns
    tw = T_TOTAL // nw
    n_chunks = tw // CHUNK
    n_groups = CHUNK // LANES
    mesh = plsc.VectorSubcoreMesh(core_axis_name="c", subcore_axis_name="s")

    @functools.partial(
        pl.kernel,
        mesh=mesh,
        out_type=[
            jax.ShapeDtypeStruct((TOP_K, T_TOTAL), jnp.int32),
            jax.ShapeDtypeStruct((TOP_K, T_TOTAL), jnp.float32),
        ],
        scratch_types=[
            pltpu.VMEM((E, CHUNK), jnp.float32),
            pltpu.VMEM((E, CHUNK), jnp.float32),
            pltpu.VMEM((TOP_K, CHUNK), jnp.int32),
            pltpu.VMEM((TOP_K, CHUNK), jnp.float32),
        ],
    )
    def sc_topk(scores_hbm, biased_hbm, outi_hbm, outw_hbm,
                scores_v, biased_v, outi_v, outw_v):
        wid = lax.axis_index("s") * nc + lax.axis_index("c")
        wbase = wid * tw

        def group_body(g, carry):
            off = g * LANES
            neg = jnp.full((LANES,), -3.0e38, jnp.float32)
            m = [neg] * TOP_K
            msc = [jnp.zeros((LANES,), jnp.float32)] * TOP_K
            mi = [jnp.zeros((LANES,), jnp.int32)] * TOP_K
            for e in range(E):
                v = biased_v[e, pl.ds(off, LANES)]
                s = scores_v[e, pl.ds(off, LANES)]
                ei = jnp.full((LANES,), e, jnp.int32)
                for j in range(TOP_K):
                    c = v > m[j]
                    nm = jnp.where(c, v, m[j])
                    ns = jnp.where(c, s, msc[j])
                    ni = jnp.where(c, ei, mi[j])
                    v = jnp.where(c, m[j], v)
                    s = jnp.where(c, msc[j], s)
                    ei = jnp.where(c, mi[j], ei)
                    m[j], msc[j], mi[j] = nm, ns, ni
            ssum = jnp.full((LANES,), 1e-20, jnp.float32)
            for j in range(TOP_K):
                ssum = ssum + msc[j]
            inv = ROUTED_SCALING_FACTOR / ssum
            for j in range(TOP_K):
                outi_v[j, pl.ds(off, LANES)] = mi[j]
                outw_v[j, pl.ds(off, LANES)] = msc[j] * inv
            return carry

        for ci in range(n_chunks):
            base = wbase + ci * CHUNK
            pltpu.sync_copy(scores_hbm.at[:, pl.ds(base, CHUNK)], scores_v)
            pltpu.sync_copy(biased_hbm.at[:, pl.ds(base, CHUNK)], biased_v)
            lax.fori_loop(0, n_groups, group_body, 0)
            pltpu.sync_copy(outi_v, outi_hbm.at[:, pl.ds(base, CHUNK)])
            pltpu.sync_copy(outw_v, outw_hbm.at[:, pl.ds(base, CHUNK)])

    return sc_topk


_sc_topk = _make_sc_topk()


@jax.jit
def _route(x, gate_weight, bias2d):
    scores, biased = _tc_gate(x, gate_weight, bias2d)
    idx_t, wgt_t = _sc_topk(scores, biased)
    return idx_t.T, wgt_t.T


def kernel(x, gate_weight, e_score_correction_bias):
    x = x.astype(jnp.float32)
    w = gate_weight.astype(jnp.float32)
    b = e_score_correction_bias.astype(jnp.float32)
    return _route(x, w, b.reshape(-1, 1))


# speedup vs baseline: 1.1193x; 1.1193x over previous
import functools

import jax
import jax.numpy as jnp
from jax import lax
from jax.experimental import pallas as pl
from jax.experimental.pallas import tpu as pltpu
from jax.experimental.pallas import tpu_sc as plsc

TOP_K = 8
ROUTED_SCALING_FACTOR = 2.5
E = 64
T_TOTAL = 32768
LANES = 16
CHUNK = 512


def _tc_stage(x_ref, w_ref, b_ref, key_ref, pay_ref):
    logits = jax.lax.dot_general(
        w_ref[...], x_ref[...], (((1,), (1,)), ((), ())),
        preferred_element_type=jnp.float32,
    )
    s = jax.nn.sigmoid(logits)
    key_ref[...] = s + b_ref[...]
    eidx = jax.lax.broadcasted_iota(jnp.int32, s.shape, 0).astype(jnp.float32)
    pay_ref[...] = s + eidx


@functools.partial(jax.jit, static_argnames=("block_t",))
def _tc_gate(x, gate_weight, bias2d, block_t=4096):
    t, d = x.shape
    e = gate_weight.shape[0]
    return pl.pallas_call(
        _tc_stage,
        grid=(t // block_t,),
        in_specs=[
            pl.BlockSpec((block_t, d), lambda i: (i, 0)),
            pl.BlockSpec((e, d), lambda i: (0, 0)),
            pl.BlockSpec((e, 1), lambda i: (0, 0)),
        ],
        out_specs=[
            pl.BlockSpec((e, block_t), lambda i: (0, i)),
            pl.BlockSpec((e, block_t), lambda i: (0, i)),
        ],
        out_shape=[
            jax.ShapeDtypeStruct((e, t), jnp.float32),
            jax.ShapeDtypeStruct((e, t), jnp.float32),
        ],
    )(x, gate_weight, bias2d)


def _make_sc_topk():
    info = plsc.get_sparse_core_info()
    nc, ns = info.num_cores, info.num_subcores
    nw = nc * ns
    tw = T_TOTAL // nw
    n_chunks = tw // CHUNK
    n_groups = CHUNK // LANES
    mesh = plsc.VectorSubcoreMesh(core_axis_name="c", subcore_axis_name="s")

    @functools.partial(
        pl.kernel,
        mesh=mesh,
        out_type=[
            jax.ShapeDtypeStruct((TOP_K, T_TOTAL), jnp.int32),
            jax.ShapeDtypeStruct((TOP_K, T_TOTAL), jnp.float32),
        ],
        scratch_types=[
            pltpu.VMEM((E, CHUNK), jnp.float32),
            pltpu.VMEM((E, CHUNK), jnp.float32),
            pltpu.VMEM((TOP_K, CHUNK), jnp.int32),
            pltpu.VMEM((TOP_K, CHUNK), jnp.float32),
        ],
    )
    def sc_topk(key_hbm, pay_hbm, outi_hbm, outw_hbm,
                key_v, pay_v, outi_v, outw_v):
        wid = lax.axis_index("s") * nc + lax.axis_index("c")
        wbase = wid * tw

        def group_body(g, carry):
            off = g * LANES
            neg = jnp.full((LANES,), -3.0e38, jnp.float32)
            m = [neg] * TOP_K
            mp = [jnp.zeros((LANES,), jnp.float32)] * TOP_K
            for e in range(E):
                v = key_v[e, pl.ds(off, LANES)]
                pv = pay_v[e, pl.ds(off, LANES)]
                for j in range(TOP_K):
                    c = v > m[j]
                    nm = jnp.where(c, v, m[j])
                    np_ = jnp.where(c, pv, mp[j])
                    v = jnp.where(c, m[j], v)
                    pv = jnp.where(c, mp[j], pv)
                    m[j], mp[j] = nm, np_
            ssum = jnp.full((LANES,), 1e-20, jnp.float32)
            idxs = []
            scs = []
            for j in range(TOP_K):
                ii = mp[j].astype(jnp.int32)
                s = mp[j] - ii.astype(jnp.float32)
                idxs.append(ii)
                scs.append(s)
                ssum = ssum + s
            inv = ROUTED_SCALING_FACTOR / ssum
            for j in range(TOP_K):
                outi_v[j, pl.ds(off, LANES)] = idxs[j]
                outw_v[j, pl.ds(off, LANES)] = scs[j] * inv
            return carry

        for ci in range(n_chunks):
            base = wbase + ci * CHUNK
            pltpu.sync_copy(key_hbm.at[:, pl.ds(base, CHUNK)], key_v)
            pltpu.sync_copy(pay_hbm.at[:, pl.ds(base, CHUNK)], pay_v)
            lax.fori_loop(0, n_groups, group_body, 0)
            pltpu.sync_copy(outi_v, outi_hbm.at[:, pl.ds(base, CHUNK)])
            pltpu.sync_copy(outw_v, outw_hbm.at[:, pl.ds(base, CHUNK)])

    return sc_topk


_sc_topk = _make_sc_topk()


@jax.jit
def _route(x, gate_weight, bias2d):
    key, pay = _tc_gate(x, gate_weight, bias2d)
    idx_t, wgt_t = _sc_topk(key, pay)
    return idx_t.T, wgt_t.T


def kernel(x, gate_weight, e_score_correction_bias):
    x = x.astype(jnp.float32)
    w = gate_weight.astype(jnp.float32)
    b = e_score_correction_bias.astype(jnp.float32)
    return _route(x, w, b.reshape(-1, 1))


# speedup vs baseline: 2.5962x; 2.3194x over previous
import functools

import jax
import jax.numpy as jnp
from jax.experimental import pallas as pl
from jax.experimental.pallas import tpu as pltpu

TOP_K = 8
ROUTED_SCALING_FACTOR = 2.5


def _gate_block(x_ref, w_ref, b_ref, idx_ref, wgt_ref):
    x = x_ref[...]
    w = w_ref[...]
    logits = jax.lax.dot_general(
        w, x, (((1,), (1,)), ((), ())), preferred_element_type=jnp.float32
    )
    scores = jax.nn.sigmoid(logits)
    biased = scores + b_ref[...]

    e, bt = scores.shape
    eidx = jax.lax.broadcasted_iota(jnp.int32, (e, bt), 0).astype(jnp.float32)
    pay = scores + eidx

    work = biased
    ssum = jnp.zeros((1, bt), jnp.float32)
    picked = []
    for k in range(TOP_K):
        mx = jnp.max(work, axis=0, keepdims=True)
        paym = jnp.min(jnp.where(work == mx, pay, 3.0e38), axis=0, keepdims=True)
        idxf = jnp.floor(paym)
        sk = paym - idxf
        idx_ref[k : k + 1, :] = idxf.astype(jnp.int32)
        picked.append(sk)
        ssum = ssum + sk
        if k + 1 < TOP_K:
            work = jnp.where(pay == paym, -jnp.inf, work)

    inv = ROUTED_SCALING_FACTOR / (ssum + 1e-20)
    wgt_ref[...] = jnp.concatenate(picked, axis=0) * inv


@functools.partial(jax.jit, static_argnames=("block_t",))
def _route(x, gate_weight, bias2d, block_t=4096):
    t, d = x.shape
    e = gate_weight.shape[0]
    grid = (t // block_t,)
    idx_t, wgt_t = pl.pallas_call(
        _gate_block,
        grid=grid,
        in_specs=[
            pl.BlockSpec((block_t, d), lambda i: (i, 0)),
            pl.BlockSpec((e, d), lambda i: (0, 0)),
            pl.BlockSpec((e, 1), lambda i: (0, 0)),
        ],
        out_specs=[
            pl.BlockSpec((TOP_K, block_t), lambda i: (0, i)),
            pl.BlockSpec((TOP_K, block_t), lambda i: (0, i)),
        ],
        out_shape=[
            jax.ShapeDtypeStruct((TOP_K, t), jnp.int32),
            jax.ShapeDtypeStruct((TOP_K, t), jnp.float32),
        ],
    )(x, gate_weight, bias2d)
    return idx_t.T, wgt_t.T


def kernel(x, gate_weight, e_score_correction_bias):
    x = x.astype(jnp.float32)
    w = gate_weight.astype(jnp.float32)
    b = e_score_correction_bias.astype(jnp.float32).reshape(-1, 1)
    idx, wgt = _route(x, w, b)
    return idx, wgt
